# baseline (speedup 1.0000x reference)
import functools

import jax
import jax.numpy as jnp
from jax import lax
from jax.experimental import pallas as pl
from jax.experimental.pallas import tpu as pltpu
from jax.experimental.pallas import tpu_sc as plsc

NC = 2
NS = 16
C = 112
DP = 128


def _prop_call(E, N):
  HALF = N // 2
  AR = HALF + NS
  ept = -(-(E // NS) // C) * C
  iters = ept // C
  iters += iters % 2
  rpt = (HALF // NS) // 8 * 8
  rlast = HALF - rpt * (NS - 1)
  zpt = (AR // NS) // 8 * 8
  zlast = AR - zpt * (NS - 1)
  mesh = plsc.VectorSubcoreMesh(core_axis_name="c", subcore_axis_name="s")

  @functools.partial(
      pl.kernel,
      out_type=jax.ShapeDtypeStruct((NC * AR, DP), jnp.float32),
      mesh=mesh,
      scratch_types=[
          pltpu.VMEM((iters, C), jnp.int32),
          pltpu.VMEM((iters, C), jnp.int32),
          pltpu.VMEM((C, DP), jnp.float32),
          pltpu.VMEM((C, DP), jnp.float32),
          pltpu.VMEM_SHARED((AR, DP), jnp.float32),
          pltpu.SemaphoreType.DMA,
          pltpu.SemaphoreType.DMA,
      ],
  )
  def prop(src_hbm, dst_hbm, g_hbm, zeros_hbm, out_hbm,
           src_v, dst_v, buf0, buf1, acc, sem0, sem1):
    c = lax.axis_index("c")
    s = lax.axis_index("s")
    row0 = pl.multiple_of(s * zpt, 8)

    @pl.when(s < NS - 1)
    def _():
      pltpu.sync_copy(zeros_hbm.at[pl.ds(row0, zpt)],
                      acc.at[pl.ds(row0, zpt)])

    @pl.when(s == NS - 1)
    def _():
      pltpu.sync_copy(zeros_hbm.at[pl.ds(zpt * (NS - 1), zlast)],
                      acc.at[pl.ds(zpt * (NS - 1), zlast)])

    base = c * HALF
    trash = HALF + s
    pltpu.sync_copy(src_hbm.at[s], src_v)
    pltpu.sync_copy(dst_hbm.at[s], dst_v)

    def fix(i, carry):
      for j in range(C // 16):
        v = dst_v[i, pl.ds(j * 16, 16)]
        t = v - base
        ok = (t >= 0) & (t < HALF)
        dst_v[i, pl.ds(j * 16, 16)] = jnp.where(ok, t, trash)
      return carry

    lax.fori_loop(0, iters, fix, 0)
    plsc.subcore_barrier()

    pltpu.async_copy(g_hbm.at[src_v.at[0]], buf0, sem0)

    def body(k, carry):
      i = 2 * k
      pltpu.async_copy(g_hbm.at[src_v.at[i + 1]], buf1, sem1)
      pltpu.make_async_copy(g_hbm.at[src_v.at[i]], buf0, sem0).wait()
      pltpu.sync_copy(buf0, acc.at[dst_v.at[i]], add=True)

      @pl.when(i + 2 < iters)
      def _():
        pltpu.async_copy(g_hbm.at[src_v.at[i + 2]], buf0, sem0)

      pltpu.make_async_copy(g_hbm.at[src_v.at[i + 1]], buf1, sem1).wait()
      pltpu.sync_copy(buf1, acc.at[dst_v.at[i + 1]], add=True)
      return carry

    lax.fori_loop(0, iters // 2, body, 0)

    plsc.subcore_barrier()
    wrow0 = pl.multiple_of(s * rpt, 8)
    cbase = pl.multiple_of(c * AR, 8)

    @pl.when(s < NS - 1)
    def _():
      pltpu.sync_copy(acc.at[pl.ds(wrow0, rpt)],
                      out_hbm.at[pl.ds(cbase + wrow0, rpt)])

    @pl.when(s == NS - 1)
    def _():
      pltpu.sync_copy(acc.at[pl.ds(rpt * (NS - 1), rlast)],
                      out_hbm.at[pl.ds(cbase + rpt * (NS - 1), rlast)])

  return prop


def _count_call(E, N):
  HALF = N // 2
  AR = HALF + NS
  ept = -(-(E // NS) // C) * C
  iters = ept // C
  iters += iters % 2
  zpt = (AR // NS) // 8 * 8
  zlast = AR - zpt * (NS - 1)
  rpt = (HALF // NS) // 8 * 8
  rlast = HALF - rpt * (NS - 1)
  mesh = plsc.VectorSubcoreMesh(core_axis_name="c", subcore_axis_name="s")

  @functools.partial(
      pl.kernel,
      out_type=jax.ShapeDtypeStruct((NC * AR, DP), jnp.float32),
      mesh=mesh,
      scratch_types=[
          pltpu.VMEM((iters, C), jnp.int32),
          pltpu.VMEM((C, DP), jnp.float32),
          pltpu.VMEM_SHARED((AR, DP), jnp.float32),
      ],
  )
  def count(dst_hbm, ones_hbm, zeros_hbm, out_hbm, dst_v, ones_v, acc):
    c = lax.axis_index("c")
    s = lax.axis_index("s")
    row0 = pl.multiple_of(s * zpt, 8)
    pltpu.sync_copy(dst_hbm.at[s], dst_v)
    pltpu.sync_copy(ones_hbm, ones_v)

    @pl.when(s < NS - 1)
    def _():
      pltpu.sync_copy(zeros_hbm.at[pl.ds(row0, zpt)],
                      acc.at[pl.ds(row0, zpt)])

    @pl.when(s == NS - 1)
    def _():
      pltpu.sync_copy(zeros_hbm.at[pl.ds(zpt * (NS - 1), zlast)],
                      acc.at[pl.ds(zpt * (NS - 1), zlast)])

    base = c * HALF
    trash = HALF + s

    def fix(i, carry):
      for j in range(C // 16):
        v = dst_v[i, pl.ds(j * 16, 16)]
        t = v - base
        ok = (t >= 0) & (t < HALF)
        dst_v[i, pl.ds(j * 16, 16)] = jnp.where(ok, t, trash)
      return carry

    lax.fori_loop(0, iters, fix, 0)
    plsc.subcore_barrier()

    def body(i, carry):
      pltpu.sync_copy(ones_v, acc.at[dst_v.at[i]], add=True)
      return carry

    lax.fori_loop(0, iters, body, 0)
    plsc.subcore_barrier()
    wrow0 = pl.multiple_of(s * rpt, 8)
    cbase = pl.multiple_of(c * AR, 8)

    @pl.when(s < NS - 1)
    def _():
      pltpu.sync_copy(acc.at[pl.ds(wrow0, rpt)],
                      out_hbm.at[pl.ds(cbase + wrow0, rpt)])

    @pl.when(s == NS - 1)
    def _():
      pltpu.sync_copy(acc.at[pl.ds(rpt * (NS - 1), rlast)],
                      out_hbm.at[pl.ds(cbase + rpt * (NS - 1), rlast)])

  return count


def _full(shape):
  return pl.BlockSpec(shape, lambda i: tuple(0 for _ in shape))


def _rows(shape):
  return pl.BlockSpec(shape, lambda i: (i,) + tuple(0 for _ in shape[1:]))


def _scale_call(N, DIN, B):
  def body(degp_ref, x_ref, g_ref, dinv_ref):
    deg = degp_ref[...] + 1.0
    dinv = lax.rsqrt(deg)
    dinv_ref[...] = dinv
    g_ref[...] = x_ref[...] * dinv

  return pl.pallas_call(
      body,
      grid=(N // B,),
      in_specs=[_rows((B, 1)), _rows((B, DIN))],
      out_specs=[_rows((B, DIN)), _rows((B, 1))],
      out_shape=[
          jax.ShapeDtypeStruct((N, DIN), jnp.float32),
          jax.ShapeDtypeStruct((N, 1), jnp.float32),
      ],
  )


def _layer1_call(N, DIN, H1, H2, B):
  def body(a, g1, dinv, W1, b1, W2, out):
    t = (a[...] + g1[...]) * dinv[...]
    h1 = jnp.maximum(
        jnp.dot(t, W1[...], preferred_element_type=jnp.float32) + b1[...], 0.0)
    g2 = jnp.dot(h1, W2[...], preferred_element_type=jnp.float32) * dinv[...]
    out[...] = jnp.concatenate(
        [g2, jnp.zeros((B, DP - H2), jnp.float32)], axis=1)

  return pl.pallas_call(
      body,
      grid=(N // B,),
      in_specs=[
          _rows((B, DIN)), _rows((B, DIN)), _rows((B, 1)),
          _full((DIN, H1)), _full((1, H1)), _full((H1, H2)),
      ],
      out_specs=_rows((B, DP)),
      out_shape=jax.ShapeDtypeStruct((N, DP), jnp.float32),
  )


def _layer2_call(N, DOUT, B):
  def body(a, g2, dinv, b2p, W3p, out):
    t = (a[...] + g2[...]) * dinv[...]
    h2 = jnp.maximum(t + b2p[...], 0.0)
    g3 = jnp.dot(h2, W3p[...], preferred_element_type=jnp.float32) * dinv[...]
    out[...] = jnp.concatenate(
        [g3, jnp.zeros((B, DP - DOUT), jnp.float32)], axis=1)

  return pl.pallas_call(
      body,
      grid=(N // B,),
      in_specs=[
          _rows((B, DP)), _rows((B, DP)), _rows((B, 1)),
          _full((1, DP)), _full((DP, DOUT)),
      ],
      out_specs=_rows((B, DP)),
      out_shape=jax.ShapeDtypeStruct((N, DP), jnp.float32),
  )


def _head_call(N, HF, B):
  def body(a, g3p, dinv, b3p, Wf1p, bf1, Wf2, bf2, out):
    t = (a[...] + g3p[...]) * dinv[...]
    h3 = jnp.maximum(t + b3p[...], 0.0)
    u = jnp.dot(h3, Wf1p[...], preferred_element_type=jnp.float32) + bf1[...]
    out[...] = jnp.dot(
        u, Wf2[...], preferred_element_type=jnp.float32) + bf2[...]

  return pl.pallas_call(
      body,
      grid=(N // B,),
      in_specs=[
          _rows((B, DP)), _rows((B, DP)), _rows((B, 1)),
          _full((1, DP)), _full((DP, HF)), _full((1, HF)),
          _full((HF, 1)), _full((1, 1)),
      ],
      out_specs=_rows((B, 1)),
      out_shape=jax.ShapeDtypeStruct((N, 1), jnp.float32),
  )


@jax.jit
def kernel(x, edge_index, W1, b1, W2, b2, W3, b3, Wf1, bf1, Wf2, bf2):
  N, DIN = x.shape
  E = edge_index.shape[1]
  H1 = W1.shape[1]
  H2 = W2.shape[1]
  DOUT = W3.shape[1]
  HF = Wf1.shape[1]
  B = 1000
  AR = N // 2 + NS
  assert DIN == DP

  ept = E // NS
  nch = -(-ept // C)
  nch += nch % 2
  ptile = nch * C
  pad = ptile - ept
  src_t = jnp.concatenate(
      [edge_index[0].reshape(NS, ept),
       jnp.zeros((NS, pad), jnp.int32)], axis=1).reshape(NS, nch, C)
  dst_t = jnp.concatenate(
      [edge_index[1].reshape(NS, ept),
       jnp.full((NS, pad), N, jnp.int32)], axis=1).reshape(NS, nch, C)
  onesC = jnp.ones((C, DP), jnp.float32)
  zac = jnp.zeros((AR, DP), jnp.float32)

  prop = _prop_call(E, N)

  def merge(o):
    return jnp.concatenate([o[:N // 2], o[AR:AR + N // 2]], axis=0)

  degp = merge(_count_call(E, N)(dst_t, onesC, zac))[:, 0:1]
  g1, dinv = _scale_call(N, DIN, B)(degp, x)

  a1 = merge(prop(src_t, dst_t, g1, zac))
  g2 = _layer1_call(N, DIN, H1, H2, B)(
      a1, g1, dinv, W1, b1.reshape(1, H1), W2)

  a2 = merge(prop(src_t, dst_t, g2, zac))
  b2p = jnp.concatenate([b2, jnp.zeros((DP - H2,), jnp.float32)]).reshape(1, DP)
  W3p = jnp.concatenate([W3, jnp.zeros((DP - H2, DOUT), jnp.float32)], axis=0)
  g3 = _layer2_call(N, DOUT, B)(a2, g2, dinv, b2p, W3p)

  a3 = merge(prop(src_t, dst_t, g3, zac))
  b3p = jnp.concatenate([b3, jnp.zeros((DP - DOUT,), jnp.float32)])
  Wf1p = jnp.concatenate([Wf1, jnp.zeros((DP - DOUT, HF), jnp.float32)], axis=0)
  y = _head_call(N, HF, B)(
      a3, g3, dinv, b3p.reshape(1, DP),
      Wf1p, bf1.reshape(1, HF), Wf2, bf2.reshape(1, 1))
  return y


# speedup vs baseline: 1.6523x; 1.6523x over previous
import functools

import jax
import jax.numpy as jnp
from jax import lax
from jax.experimental import pallas as pl
from jax.experimental.pallas import tpu as pltpu
from jax.experimental.pallas import tpu_sc as plsc

NC = 2
NS = 16
C = 80
DP = 128


def _prop_call(E, N):
  HALF = N // 2
  AR = HALF + NS
  ept = -(-(E // NS) // C) * C
  iters = ept // C
  iters += iters % 2
  rpt = (HALF // NS) // 8 * 8
  rlast = HALF - rpt * (NS - 1)
  zpt = (AR // NS) // 8 * 8
  zlast = AR - zpt * (NS - 1)
  mesh = plsc.VectorSubcoreMesh(core_axis_name="c", subcore_axis_name="s")

  @functools.partial(
      pl.kernel,
      out_type=jax.ShapeDtypeStruct((NC * AR, DP), jnp.float32),
      mesh=mesh,
      scratch_types=[
          pltpu.VMEM((iters, C), jnp.int32),
          pltpu.VMEM((iters, C), jnp.int32),
          pltpu.VMEM((C, DP), jnp.float32),
          pltpu.VMEM((C, DP), jnp.float32),
          pltpu.VMEM_SHARED((AR, DP), jnp.float32),
          pltpu.SemaphoreType.DMA,
          pltpu.SemaphoreType.DMA,
      ],
  )
  def prop(src_hbm, dst_hbm, g_hbm, zeros_hbm, out_hbm,
           src_v, dst_v, buf0, buf1, acc, sem0, sem1):
    c = lax.axis_index("c")
    s = lax.axis_index("s")
    row0 = pl.multiple_of(s * zpt, 8)

    @pl.when(s < NS - 1)
    def _():
      pltpu.sync_copy(zeros_hbm.at[pl.ds(row0, zpt)],
                      acc.at[pl.ds(row0, zpt)])

    @pl.when(s == NS - 1)
    def _():
      pltpu.sync_copy(zeros_hbm.at[pl.ds(zpt * (NS - 1), zlast)],
                      acc.at[pl.ds(zpt * (NS - 1), zlast)])

    base = c * HALF
    trash = HALF + s
    pltpu.sync_copy(src_hbm.at[s], src_v)
    pltpu.sync_copy(dst_hbm.at[s], dst_v)

    def fix(i, carry):
      for j in range(C // 16):
        v = dst_v[i, pl.ds(j * 16, 16)]
        t = v - base
        ok = (t >= 0) & (t < HALF)
        dst_v[i, pl.ds(j * 16, 16)] = jnp.where(ok, t, trash)
      return carry

    lax.fori_loop(0, iters, fix, 0)
    plsc.subcore_barrier()

    pltpu.async_copy(g_hbm.at[src_v.at[0]], buf0, sem0)

    def body(k, carry):
      i = 2 * k
      pltpu.async_copy(g_hbm.at[src_v.at[i + 1]], buf1, sem1)
      pltpu.make_async_copy(g_hbm.at[src_v.at[i]], buf0, sem0).wait()
      pltpu.sync_copy(buf0, acc.at[dst_v.at[i]], add=True)

      @pl.when(i + 2 < iters)
      def _():
        pltpu.async_copy(g_hbm.at[src_v.at[i + 2]], buf0, sem0)

      pltpu.make_async_copy(g_hbm.at[src_v.at[i + 1]], buf1, sem1).wait()
      pltpu.sync_copy(buf1, acc.at[dst_v.at[i + 1]], add=True)
      return carry

    lax.fori_loop(0, iters // 2, body, 0)

    plsc.subcore_barrier()
    wrow0 = pl.multiple_of(s * rpt, 8)
    cbase = pl.multiple_of(c * AR, 8)

    @pl.when(s < NS - 1)
    def _():
      pltpu.sync_copy(acc.at[pl.ds(wrow0, rpt)],
                      out_hbm.at[pl.ds(cbase + wrow0, rpt)])

    @pl.when(s == NS - 1)
    def _():
      pltpu.sync_copy(acc.at[pl.ds(rpt * (NS - 1), rlast)],
                      out_hbm.at[pl.ds(cbase + rpt * (NS - 1), rlast)])

  return prop


def _count_call(E, N):
  EPW = E // (NC * NS)
  iters = -(-EPW // C)
  zpt = (N // NS) // 8 * 8
  zlast = N - zpt * (NS - 1)
  mesh = plsc.VectorSubcoreMesh(core_axis_name="c", subcore_axis_name="s")

  @functools.partial(
      pl.kernel,
      out_type=jax.ShapeDtypeStruct((NC * N, DP), jnp.float32),
      mesh=mesh,
      scratch_types=[
          pltpu.VMEM((iters, C), jnp.int32),
          pltpu.VMEM((C, DP), jnp.float32),
          pltpu.VMEM_SHARED((N, DP), jnp.float32),
      ],
  )
  def count(dst_hbm, ones_hbm, zeros_hbm, out_hbm, dst_v, ones_v, acc):
    c = lax.axis_index("c")
    s = lax.axis_index("s")
    row0 = pl.multiple_of(s * zpt, 8)
    pltpu.sync_copy(dst_hbm.at[c * NS + s], dst_v)
    pltpu.sync_copy(ones_hbm, ones_v)

    @pl.when(s < NS - 1)
    def _():
      pltpu.sync_copy(zeros_hbm.at[pl.ds(row0, zpt)],
                      acc.at[pl.ds(row0, zpt)])

    @pl.when(s == NS - 1)
    def _():
      pltpu.sync_copy(zeros_hbm.at[pl.ds(zpt * (NS - 1), zlast)],
                      acc.at[pl.ds(zpt * (NS - 1), zlast)])

    plsc.subcore_barrier()

    def body(i, carry):
      pltpu.sync_copy(ones_v, acc.at[dst_v.at[i]], add=True)
      return carry

    lax.fori_loop(0, iters, body, 0)
    plsc.subcore_barrier()
    cbase = pl.multiple_of(c * N, 8)

    @pl.when(s < NS - 1)
    def _():
      pltpu.sync_copy(acc.at[pl.ds(row0, zpt)],
                      out_hbm.at[pl.ds(cbase + row0, zpt)])

    @pl.when(s == NS - 1)
    def _():
      pltpu.sync_copy(acc.at[pl.ds(zpt * (NS - 1), zlast)],
                      out_hbm.at[pl.ds(cbase + zpt * (NS - 1), zlast)])

  return count


def _full(shape):
  return pl.BlockSpec(shape, lambda i: tuple(0 for _ in shape))


def _rows(shape):
  return pl.BlockSpec(shape, lambda i: (i,) + tuple(0 for _ in shape[1:]))


def _scale_call(N, DIN, B):
  def body(degp_ref, x_ref, g_ref, dinv_ref):
    deg = degp_ref[...] + 1.0
    dinv = lax.rsqrt(deg)
    dinv_ref[...] = dinv
    g_ref[...] = x_ref[...] * dinv

  return pl.pallas_call(
      body,
      grid=(N // B,),
      in_specs=[_rows((B, 1)), _rows((B, DIN))],
      out_specs=[_rows((B, DIN)), _rows((B, 1))],
      out_shape=[
          jax.ShapeDtypeStruct((N, DIN), jnp.float32),
          jax.ShapeDtypeStruct((N, 1), jnp.float32),
      ],
  )


def _layer1_call(N, DIN, H1, H2, B):
  def body(a, g1, dinv, W1, b1, W2, out):
    t = (a[...] + g1[...]) * dinv[...]
    h1 = jnp.maximum(
        jnp.dot(t, W1[...], preferred_element_type=jnp.float32) + b1[...], 0.0)
    g2 = jnp.dot(h1, W2[...], preferred_element_type=jnp.float32) * dinv[...]
    out[...] = jnp.concatenate(
        [g2, jnp.zeros((B, DP - H2), jnp.float32)], axis=1)

  return pl.pallas_call(
      body,
      grid=(N // B,),
      in_specs=[
          _rows((B, DIN)), _rows((B, DIN)), _rows((B, 1)),
          _full((DIN, H1)), _full((1, H1)), _full((H1, H2)),
      ],
      out_specs=_rows((B, DP)),
      out_shape=jax.ShapeDtypeStruct((N, DP), jnp.float32),
  )


def _layer2_call(N, DOUT, B):
  def body(a, g2, dinv, b2p, W3p, out):
    t = (a[...] + g2[...]) * dinv[...]
    h2 = jnp.maximum(t + b2p[...], 0.0)
    g3 = jnp.dot(h2, W3p[...], preferred_element_type=jnp.float32) * dinv[...]
    out[...] = jnp.concatenate(
        [g3, jnp.zeros((B, DP - DOUT), jnp.float32)], axis=1)

  return pl.pallas_call(
      body,
      grid=(N // B,),
      in_specs=[
          _rows((B, DP)), _rows((B, DP)), _rows((B, 1)),
          _full((1, DP)), _full((DP, DOUT)),
      ],
      out_specs=_rows((B, DP)),
      out_shape=jax.ShapeDtypeStruct((N, DP), jnp.float32),
  )


def _head_call(N, HF, B):
  def body(a, g3p, dinv, b3p, Wf1p, bf1, Wf2, bf2, out):
    t = (a[...] + g3p[...]) * dinv[...]
    h3 = jnp.maximum(t + b3p[...], 0.0)
    u = jnp.dot(h3, Wf1p[...], preferred_element_type=jnp.float32) + bf1[...]
    out[...] = jnp.dot(
        u, Wf2[...], preferred_element_type=jnp.float32) + bf2[...]

  return pl.pallas_call(
      body,
      grid=(N // B,),
      in_specs=[
          _rows((B, DP)), _rows((B, DP)), _rows((B, 1)),
          _full((1, DP)), _full((DP, HF)), _full((1, HF)),
          _full((HF, 1)), _full((1, 1)),
      ],
      out_specs=_rows((B, 1)),
      out_shape=jax.ShapeDtypeStruct((N, 1), jnp.float32),
  )


@jax.jit
def kernel(x, edge_index, W1, b1, W2, b2, W3, b3, Wf1, bf1, Wf2, bf2):
  N, DIN = x.shape
  E = edge_index.shape[1]
  H1 = W1.shape[1]
  H2 = W2.shape[1]
  DOUT = W3.shape[1]
  HF = Wf1.shape[1]
  B = 1000
  AR = N // 2 + NS
  assert DIN == DP

  ept = E // NS
  nch = -(-ept // C)
  nch += nch % 2
  ptile = nch * C
  pad = ptile - ept
  src_t = jnp.concatenate(
      [edge_index[0].reshape(NS, ept),
       jnp.zeros((NS, pad), jnp.int32)], axis=1).reshape(NS, nch, C)
  dst_t = jnp.concatenate(
      [edge_index[1].reshape(NS, ept),
       jnp.full((NS, pad), N, jnp.int32)], axis=1).reshape(NS, nch, C)
  onesC = jnp.ones((C, DP), jnp.float32)
  zac = jnp.zeros((AR, DP), jnp.float32)

  prop = _prop_call(E, N)

  def merge(o):
    return jnp.concatenate([o[:N // 2], o[AR:AR + N // 2]], axis=0)

  dst_w = edge_index[1].reshape(NC * NS, E // (NC * NS * C), C)
  zn = jnp.zeros((N, DP), jnp.float32)
  cnt = _count_call(E, N)(dst_w, onesC, zn)
  degp = (cnt[:N, 0:1] + cnt[N:, 0:1])
  g1, dinv = _scale_call(N, DIN, B)(degp, x)

  a1 = merge(prop(src_t, dst_t, g1, zac))
  g2 = _layer1_call(N, DIN, H1, H2, B)(
      a1, g1, dinv, W1, b1.reshape(1, H1), W2)

  a2 = merge(prop(src_t, dst_t, g2, zac))
  b2p = jnp.concatenate([b2, jnp.zeros((DP - H2,), jnp.float32)]).reshape(1, DP)
  W3p = jnp.concatenate([W3, jnp.zeros((DP - H2, DOUT), jnp.float32)], axis=0)
  g3 = _layer2_call(N, DOUT, B)(a2, g2, dinv, b2p, W3p)

  a3 = merge(prop(src_t, dst_t, g3, zac))
  b3p = jnp.concatenate([b3, jnp.zeros((DP - DOUT,), jnp.float32)])
  Wf1p = jnp.concatenate([Wf1, jnp.zeros((DP - DOUT, HF), jnp.float32)], axis=0)
  y = _head_call(N, HF, B)(
      a3, g3, dinv, b3p.reshape(1, DP),
      Wf1p, bf1.reshape(1, HF), Wf2, bf2.reshape(1, 1))
  return y
